# baseline (speedup 1.0000x reference)
import functools
import jax
import jax.numpy as jnp
from jax import lax
from jax.experimental import pallas as pl
from jax.experimental.pallas import tpu as pltpu, tpu_sc as plsc

_K = 8
_LAMBDA_TEXT = 0.5
_LAMBDA_STICK = 2.0
_LAMBDA_SMOOTH = 0.1

_NC, _NS, _L = 2, 16, 16
_NW = _NC * _NS

_TC_SMOOTH_ROWS = 1536
_R = 256


def _tc_body(p_rows_ref, pt_ref, p0t_ref, p41_ref, z3_ref, zt_ref,
             fwd_ref, bwd_ref, smooth_ref, text_ref, colmin_ref):
    t = pl.program_id(1)
    T = pl.num_programs(1)
    p_tile = p_rows_ref[0]
    pt = pt_ref[0]
    p0t = p0t_ref[0]
    p41 = p41_ref[0]
    R = p_tile.shape[0]
    N = pt.shape[1]

    px = p_tile[:, 0:1]
    py = p_tile[:, 1:2]
    pz = p_tile[:, 2:3]
    dx = px - p0t[0:1, :]
    dy = py - p0t[1:2, :]
    dz = pz - p0t[2:3, :]
    d0 = dx * dx + dy * dy + dz * dz
    fwd_ref[0, 0, 0, 0] = jnp.sum(jnp.min(d0, axis=1))
    cm = jnp.min(d0, axis=0, keepdims=True)

    @pl.when(t == 0)
    def _():
        colmin_ref[...] = cm

    @pl.when(t > 0)
    def _():
        colmin_ref[...] = jnp.minimum(colmin_ref[...], cm)

    bwd_ref[0, 0, 0, 0] = jnp.where(t == T - 1, jnp.sum(colmin_ref[...]), 0.0)

    smooth_ref[0, 0, 0, 0] = 0.0

    @pl.when(t < _TC_SMOOTH_ROWS // R)
    def _():
        sx = px - pt[0:1, :]
        sy = py - pt[1:2, :]
        sz = pz - pt[2:3, :]
        ds = sx * sx + sy * sy + sz * sz
        row_ids = t * R + lax.broadcasted_iota(jnp.int32, (R, N), 0)
        col_ids = lax.broadcasted_iota(jnp.int32, (R, N), 1)
        ds = jnp.where(row_ids == col_ids, ds + 1.0e6, ds)

        work = ds
        for _ in range(_K - 1):
            m = jnp.min(work, axis=1, keepdims=True)
            work = jnp.where(work <= m, jnp.inf, work)
        t8 = jnp.min(work, axis=1, keepdims=True)

        maskf = (ds <= t8).astype(jnp.float32)
        nb = lax.dot_general(maskf, p41, (((1,), (0,)), ((), ())),
                             precision=lax.Precision.HIGHEST,
                             preferred_element_type=jnp.float32)
        count = nb[:, 3:4]
        ex = px - nb[:, 0:1] / count
        ey = py - nb[:, 1:2] / count
        ez = pz - nb[:, 2:3] / count
        smooth_ref[0, 0, 0, 0] = jnp.sum(ex * ex + ey * ey + ez * ez)

    z3 = z3_ref[...]
    zt = zt_ref[...]
    n3 = jnp.maximum(jnp.sqrt(jnp.sum(z3 * z3)), 1.0e-12)
    nt = jnp.maximum(jnp.sqrt(jnp.sum(zt * zt)), 1.0e-12)
    text_ref[0, 0, 0, 0] = jnp.sum(z3 * zt) / (n3 * nt)


def _tc_part(P, PT, P0, z_3d, z_text):
    B, N, _ = P.shape
    T = N // _R
    P0T = P0.transpose(0, 2, 1)
    P41 = jnp.concatenate([P, jnp.ones((B, N, 1), jnp.float32)], axis=2)
    out_shapes = [jax.ShapeDtypeStruct((B, T, 1, 1), jnp.float32)] * 4
    scalar_spec = pl.BlockSpec((1, 1, 1, 1), lambda b, t: (b, t, 0, 0),
                               memory_space=pltpu.SMEM)
    return pl.pallas_call(
        _tc_body,
        grid=(B, T),
        in_specs=[
            pl.BlockSpec((1, _R, 3), lambda b, t: (b, t, 0)),
            pl.BlockSpec((1, 3, N), lambda b, t: (b, 0, 0)),
            pl.BlockSpec((1, 3, N), lambda b, t: (b, 0, 0)),
            pl.BlockSpec((1, N, 4), lambda b, t: (b, 0, 0)),
            pl.BlockSpec((1, 1, z_3d.shape[1]), lambda b, t: (b, 0, 0)),
            pl.BlockSpec((1, 1, z_text.shape[1]), lambda b, t: (b, 0, 0)),
        ],
        out_specs=[scalar_spec] * 4,
        out_shape=out_shapes,
        scratch_shapes=[pltpu.VMEM((1, N), jnp.float32)],
        compiler_params=pltpu.CompilerParams(
            dimension_semantics=("arbitrary", "arbitrary"),
        ),
    )(P, PT, P0T, P41, z_3d[:, None, :], z_text[:, None, :])


def _make_sc_smooth(B, N):
    M = N - _TC_SMOOTH_ROWS
    PPB = _NW // B
    RPW = M // PPB
    CH = N // _L
    mesh = plsc.VectorSubcoreMesh(core_axis_name="c", subcore_axis_name="s")

    @functools.partial(
        pl.kernel,
        out_type=jax.ShapeDtypeStruct((_NW, _L), jnp.float32),
        mesh=mesh,
        scratch_types=[
            pltpu.VMEM((N,), jnp.float32),
            pltpu.VMEM((N,), jnp.float32),
            pltpu.VMEM((N,), jnp.float32),
            pltpu.VMEM((CH, _L), jnp.float32),
            pltpu.VMEM((_L,), jnp.float32),
        ],
        compiler_params=pltpu.CompilerParams(needs_layout_passes=False),
    )
    def sc_smooth(pt_hbm, out_hbm, pxv, pyv, pzv, dbuf, accv):
        wid = lax.axis_index("s") * _NC + lax.axis_index("c")
        b = wid // PPB
        part = wid % PPB
        pltpu.sync_copy(pt_hbm.at[b * 3], pxv)
        pltpu.sync_copy(pt_hbm.at[b * 3 + 1], pyv)
        pltpu.sync_copy(pt_hbm.at[b * 3 + 2], pzv)
        base = _TC_SMOOTH_ROWS + part * RPW

        lane_ids = jnp.arange(_L, dtype=jnp.int32)
        inf = jnp.full((_L,), jnp.inf)

        def row_body(r, acc):
            g0 = base + (r // _L) * _L
            onehot = lane_ids == jnp.full((_L,), r % _L, dtype=jnp.int32)
            gx = pxv[pl.ds(g0, _L)]
            gy = pyv[pl.ds(g0, _L)]
            gz = pzv[pl.ds(g0, _L)]
            qx = jnp.full((_L,), jnp.sum(jnp.where(onehot, gx, 0.0)))
            qy = jnp.full((_L,), jnp.sum(jnp.where(onehot, gy, 0.0)))
            qz = jnp.full((_L,), jnp.sum(jnp.where(onehot, gz, 0.0)))

            pxv[pl.ds(g0, _L)] = jnp.where(onehot, inf, gx)

            def ch_body(c, ts):
                kx = pxv[pl.ds(c * _L, _L)]
                ky = pyv[pl.ds(c * _L, _L)]
                kz = pzv[pl.ds(c * _L, _L)]
                dx = qx - kx
                dy = qy - ky
                dz = qz - kz
                d = dx * dx + dy * dy + dz * dz
                dbuf[c] = d
                x = d
                out = []
                for t in ts:
                    lo = jnp.minimum(t, x)
                    x = jnp.maximum(t, x)
                    out.append(lo)
                return tuple(out)

            ts = lax.fori_loop(0, CH, ch_body, (inf,) * _K)

            work = list(ts)
            th = None
            for i in range(_K):
                m = work[0]
                for w in work[1:]:
                    m = jnp.minimum(m, w)
                s = jnp.min(m)
                if i < _K - 1:
                    sv = jnp.full((_L,), s)
                    work = [jnp.where(w <= sv, jnp.inf, w) for w in work]
                else:
                    th = s
            thv = jnp.full((_L,), th)

            def p2_body(c, carry):
                sx, sy, sz, cnt = carry
                d = dbuf[c]
                kx = pxv[pl.ds(c * _L, _L)]
                ky = pyv[pl.ds(c * _L, _L)]
                kz = pzv[pl.ds(c * _L, _L)]
                msk = d <= thv
                sx = sx + jnp.where(msk, kx, 0.0)
                sy = sy + jnp.where(msk, ky, 0.0)
                sz = sz + jnp.where(msk, kz, 0.0)
                cnt = cnt + jnp.where(msk, 1.0, 0.0)
                return (sx, sy, sz, cnt)

            z = jnp.zeros((_L,), jnp.float32)
            sx, sy, sz, cnt = lax.fori_loop(0, CH, p2_body, (z, z, z, z))
            pxv[pl.ds(g0, _L)] = gx

            csv = jnp.full((_L,), jnp.sum(cnt))
            ex = qx - jnp.full((_L,), jnp.sum(sx)) / csv
            ey = qy - jnp.full((_L,), jnp.sum(sy)) / csv
            ez = qz - jnp.full((_L,), jnp.sum(sz)) / csv
            return acc + ex * ex + ey * ey + ez * ez

        acc = lax.fori_loop(0, RPW, row_body, jnp.zeros((_L,), jnp.float32))
        accv[...] = acc
        pltpu.sync_copy(accv, out_hbm.at[wid])

    return sc_smooth


@jax.jit
def kernel(P, P0, z_3d, z_text):
    B, N, _ = P.shape
    PT = P.transpose(0, 2, 1)
    fwd, bwd, smooth_tc, text = _tc_part(P, PT, P0, z_3d, z_text)
    sc_out = _make_sc_smooth(B, N)(PT.reshape(B * 3, N))

    inv = 1.0 / (B * N)
    L_stick = (jnp.sum(fwd) + jnp.sum(bwd)) * inv
    L_smooth = (jnp.sum(smooth_tc) + jnp.sum(sc_out[:, 0])) * (inv / 3.0)
    L_text = -jnp.mean(text[:, 0, 0, 0])
    L_total = (_LAMBDA_TEXT * L_text + _LAMBDA_STICK * L_stick
               + _LAMBDA_SMOOTH * L_smooth)
    return jnp.stack([L_total, L_text, L_stick, L_smooth])


# speedup vs baseline: 1.2305x; 1.2305x over previous
import functools
import jax
import jax.numpy as jnp
from jax import lax
from jax.experimental import pallas as pl
from jax.experimental.pallas import tpu as pltpu, tpu_sc as plsc

_K = 8
_LAMBDA_TEXT = 0.5
_LAMBDA_STICK = 2.0
_LAMBDA_SMOOTH = 0.1

_NC, _NS, _L = 2, 16, 16
_NW = _NC * _NS

_TC_SMOOTH_ROWS = 1536
_R = 256


def _tc_body(p_rows_ref, pt_ref, p0t_ref, z3_ref, zt_ref,
             fwd_ref, bwd_ref, smooth_ref, text_ref, colmin_ref):
    t = pl.program_id(1)
    T = pl.num_programs(1)
    p_tile = p_rows_ref[0]
    pt = pt_ref[0]
    p0t = p0t_ref[0]
    R = p_tile.shape[0]
    N = pt.shape[1]

    px = p_tile[:, 0:1]
    py = p_tile[:, 1:2]
    pz = p_tile[:, 2:3]
    dx = px - p0t[0:1, :]
    dy = py - p0t[1:2, :]
    dz = pz - p0t[2:3, :]
    d0 = dx * dx + dy * dy + dz * dz
    fwd_ref[0, 0, 0, 0] = jnp.sum(jnp.min(d0, axis=1))
    cm = jnp.min(d0, axis=0, keepdims=True)

    @pl.when(t == 0)
    def _():
        colmin_ref[...] = cm

    @pl.when(t > 0)
    def _():
        colmin_ref[...] = jnp.minimum(colmin_ref[...], cm)

    bwd_ref[0, 0, 0, 0] = jnp.where(t == T - 1, jnp.sum(colmin_ref[...]), 0.0)

    smooth_ref[0, 0, 0, 0] = 0.0

    @pl.when(t < _TC_SMOOTH_ROWS // R)
    def _():
        sx = px - pt[0:1, :]
        sy = py - pt[1:2, :]
        sz = pz - pt[2:3, :]
        ds = sx * sx + sy * sy + sz * sz
        row_ids = t * R + lax.broadcasted_iota(jnp.int32, (R, N), 0)
        col_ids = lax.broadcasted_iota(jnp.int32, (R, N), 1)
        ds = jnp.where(row_ids == col_ids, ds + 1.0e6, ds)

        work = ds
        for _ in range(_K - 1):
            m = jnp.min(work, axis=1, keepdims=True)
            work = jnp.where(work <= m, jnp.inf, work)
        t8 = jnp.min(work, axis=1, keepdims=True)

        maskf = (ds <= t8).astype(jnp.float32)
        count = jnp.sum(maskf, axis=1, keepdims=True)
        nx = jnp.sum(maskf * pt[0:1, :], axis=1, keepdims=True)
        ny = jnp.sum(maskf * pt[1:2, :], axis=1, keepdims=True)
        nz = jnp.sum(maskf * pt[2:3, :], axis=1, keepdims=True)
        ex = px - nx / count
        ey = py - ny / count
        ez = pz - nz / count
        smooth_ref[0, 0, 0, 0] = jnp.sum(ex * ex + ey * ey + ez * ez)

    z3 = z3_ref[...]
    zt = zt_ref[...]
    n3 = jnp.maximum(jnp.sqrt(jnp.sum(z3 * z3)), 1.0e-12)
    nt = jnp.maximum(jnp.sqrt(jnp.sum(zt * zt)), 1.0e-12)
    text_ref[0, 0, 0, 0] = jnp.sum(z3 * zt) / (n3 * nt)


def _tc_part(P, PT, P0, z_3d, z_text):
    B, N, _ = P.shape
    T = N // _R
    P0T = P0.transpose(0, 2, 1)
    out_shapes = [jax.ShapeDtypeStruct((B, T, 1, 1), jnp.float32)] * 4
    scalar_spec = pl.BlockSpec((1, 1, 1, 1), lambda b, t: (b, t, 0, 0),
                               memory_space=pltpu.SMEM)
    return pl.pallas_call(
        _tc_body,
        grid=(B, T),
        in_specs=[
            pl.BlockSpec((1, _R, 3), lambda b, t: (b, t, 0)),
            pl.BlockSpec((1, 3, N), lambda b, t: (b, 0, 0)),
            pl.BlockSpec((1, 3, N), lambda b, t: (b, 0, 0)),
            pl.BlockSpec((1, 1, z_3d.shape[1]), lambda b, t: (b, 0, 0)),
            pl.BlockSpec((1, 1, z_text.shape[1]), lambda b, t: (b, 0, 0)),
        ],
        out_specs=[scalar_spec] * 4,
        out_shape=out_shapes,
        scratch_shapes=[pltpu.VMEM((1, N), jnp.float32)],
        compiler_params=pltpu.CompilerParams(
            dimension_semantics=("arbitrary", "arbitrary"),
        ),
    )(P, PT, P0T, z_3d[:, None, :], z_text[:, None, :])


def _make_sc_smooth(B, N):
    M = N - _TC_SMOOTH_ROWS
    PPB = _NW // B
    RPW = M // PPB
    CH = N // _L
    mesh = plsc.VectorSubcoreMesh(core_axis_name="c", subcore_axis_name="s")

    @functools.partial(
        pl.kernel,
        out_type=jax.ShapeDtypeStruct((_NW, _L), jnp.float32),
        mesh=mesh,
        scratch_types=[
            pltpu.VMEM((N,), jnp.float32),
            pltpu.VMEM((N,), jnp.float32),
            pltpu.VMEM((N,), jnp.float32),
            pltpu.VMEM((CH, _L), jnp.float32),
            pltpu.VMEM((_L,), jnp.float32),
        ],
        compiler_params=pltpu.CompilerParams(needs_layout_passes=False),
    )
    def sc_smooth(pt_hbm, out_hbm, pxv, pyv, pzv, dbuf, accv):
        wid = lax.axis_index("s") * _NC + lax.axis_index("c")
        b = wid // PPB
        part = wid % PPB
        pltpu.sync_copy(pt_hbm.at[b * 3], pxv)
        pltpu.sync_copy(pt_hbm.at[b * 3 + 1], pyv)
        pltpu.sync_copy(pt_hbm.at[b * 3 + 2], pzv)
        base = _TC_SMOOTH_ROWS + part * RPW

        lane_ids = jnp.arange(_L, dtype=jnp.int32)
        inf = jnp.full((_L,), jnp.inf)

        def row_body(r, acc):
            g0 = base + (r // _L) * _L
            onehot = lane_ids == jnp.full((_L,), r % _L, dtype=jnp.int32)
            gx = pxv[pl.ds(g0, _L)]
            gy = pyv[pl.ds(g0, _L)]
            gz = pzv[pl.ds(g0, _L)]
            qx = jnp.full((_L,), jnp.sum(jnp.where(onehot, gx, 0.0)))
            qy = jnp.full((_L,), jnp.sum(jnp.where(onehot, gy, 0.0)))
            qz = jnp.full((_L,), jnp.sum(jnp.where(onehot, gz, 0.0)))

            pxv[pl.ds(g0, _L)] = jnp.where(onehot, inf, gx)

            def ch_body(c, ts):
                kx = pxv[pl.ds(c * _L, _L)]
                ky = pyv[pl.ds(c * _L, _L)]
                kz = pzv[pl.ds(c * _L, _L)]
                dx = qx - kx
                dy = qy - ky
                dz = qz - kz
                d = dx * dx + dy * dy + dz * dz
                dbuf[c] = d
                x = d
                out = []
                for t in ts:
                    lo = jnp.minimum(t, x)
                    x = jnp.maximum(t, x)
                    out.append(lo)
                return tuple(out)

            ts = lax.fori_loop(0, CH, ch_body, (inf,) * _K)

            work = list(ts)
            th = None
            for i in range(_K):
                m = work[0]
                for w in work[1:]:
                    m = jnp.minimum(m, w)
                s = jnp.min(m)
                if i < _K - 1:
                    sv = jnp.full((_L,), s)
                    work = [jnp.where(w <= sv, jnp.inf, w) for w in work]
                else:
                    th = s
            thv = jnp.full((_L,), th)

            def p2_body(c, carry):
                sx, sy, sz, cnt = carry
                d = dbuf[c]
                kx = pxv[pl.ds(c * _L, _L)]
                ky = pyv[pl.ds(c * _L, _L)]
                kz = pzv[pl.ds(c * _L, _L)]
                msk = d <= thv
                sx = sx + jnp.where(msk, kx, 0.0)
                sy = sy + jnp.where(msk, ky, 0.0)
                sz = sz + jnp.where(msk, kz, 0.0)
                cnt = cnt + jnp.where(msk, 1.0, 0.0)
                return (sx, sy, sz, cnt)

            z = jnp.zeros((_L,), jnp.float32)
            sx, sy, sz, cnt = lax.fori_loop(0, CH, p2_body, (z, z, z, z))
            pxv[pl.ds(g0, _L)] = gx

            csv = jnp.full((_L,), jnp.sum(cnt))
            ex = qx - jnp.full((_L,), jnp.sum(sx)) / csv
            ey = qy - jnp.full((_L,), jnp.sum(sy)) / csv
            ez = qz - jnp.full((_L,), jnp.sum(sz)) / csv
            return acc + ex * ex + ey * ey + ez * ez

        acc = lax.fori_loop(0, RPW, row_body, jnp.zeros((_L,), jnp.float32))
        accv[...] = acc
        pltpu.sync_copy(accv, out_hbm.at[wid])

    return sc_smooth


@jax.jit
def kernel(P, P0, z_3d, z_text):
    B, N, _ = P.shape
    PT = P.transpose(0, 2, 1)
    fwd, bwd, smooth_tc, text = _tc_part(P, PT, P0, z_3d, z_text)
    sc_out = _make_sc_smooth(B, N)(PT.reshape(B * 3, N))

    inv = 1.0 / (B * N)
    L_stick = (jnp.sum(fwd) + jnp.sum(bwd)) * inv
    L_smooth = (jnp.sum(smooth_tc) + jnp.sum(sc_out[:, 0])) * (inv / 3.0)
    L_text = -jnp.mean(text[:, 0, 0, 0])
    L_total = (_LAMBDA_TEXT * L_text + _LAMBDA_STICK * L_stick
               + _LAMBDA_SMOOTH * L_smooth)
    return jnp.stack([L_total, L_text, L_stick, L_smooth])


# speedup vs baseline: 1.3543x; 1.1006x over previous
import functools
import jax
import jax.numpy as jnp
from jax import lax
from jax.experimental import pallas as pl
from jax.experimental.pallas import tpu as pltpu, tpu_sc as plsc

_K = 8
_LAMBDA_TEXT = 0.5
_LAMBDA_STICK = 2.0
_LAMBDA_SMOOTH = 0.1

_NC, _NS, _L = 2, 16, 16
_NW = _NC * _NS

_TC_SMOOTH_ROWS = 1536
_R = 256


def _tc_body(p_rows_ref, pt_ref, p0t_ref, z3_ref, zt_ref,
             out_ref, colmin_ref, acc_ref):
    b = pl.program_id(0)
    t = pl.program_id(1)
    T = pl.num_programs(1)
    B = pl.num_programs(0)
    p_tile = p_rows_ref[0]
    pt = pt_ref[0]
    p0t = p0t_ref[0]
    R = p_tile.shape[0]
    N = pt.shape[1]

    px = p_tile[:, 0:1]
    py = p_tile[:, 1:2]
    pz = p_tile[:, 2:3]
    dx = px - p0t[0:1, :]
    dy = py - p0t[1:2, :]
    dz = pz - p0t[2:3, :]
    d0 = dx * dx + dy * dy + dz * dz

    @pl.when((b == 0) & (t == 0))
    def _():
        acc_ref[0] = 0.0
        acc_ref[1] = 0.0
        acc_ref[2] = 0.0
        acc_ref[3] = 0.0

    acc_ref[0] += jnp.sum(jnp.min(d0, axis=1))
    cm = jnp.min(d0, axis=0, keepdims=True)

    @pl.when(t == 0)
    def _():
        colmin_ref[...] = cm

    @pl.when(t > 0)
    def _():
        colmin_ref[...] = jnp.minimum(colmin_ref[...], cm)

    @pl.when(t == T - 1)
    def _():
        acc_ref[1] += jnp.sum(colmin_ref[...])

    @pl.when(t < _TC_SMOOTH_ROWS // R)
    def _():
        sx = px - pt[0:1, :]
        sy = py - pt[1:2, :]
        sz = pz - pt[2:3, :]
        ds = sx * sx + sy * sy + sz * sz
        row_ids = t * R + lax.broadcasted_iota(jnp.int32, (R, N), 0)
        col_ids = lax.broadcasted_iota(jnp.int32, (R, N), 1)
        ds = jnp.where(row_ids == col_ids, ds + 1.0e6, ds)

        work = ds
        for _ in range(_K - 1):
            m = jnp.min(work, axis=1, keepdims=True)
            work = jnp.where(work <= m, jnp.inf, work)
        t8 = jnp.min(work, axis=1, keepdims=True)

        maskf = (ds <= t8).astype(jnp.float32)
        count = jnp.sum(maskf, axis=1, keepdims=True)
        nx = jnp.sum(maskf * pt[0:1, :], axis=1, keepdims=True)
        ny = jnp.sum(maskf * pt[1:2, :], axis=1, keepdims=True)
        nz = jnp.sum(maskf * pt[2:3, :], axis=1, keepdims=True)
        ex = px - nx / count
        ey = py - ny / count
        ez = pz - nz / count
        acc_ref[2] += jnp.sum(ex * ex + ey * ey + ez * ez)

    @pl.when(t == 0)
    def _():
        z3 = z3_ref[...]
        zt = zt_ref[...]
        n3 = jnp.maximum(jnp.sqrt(jnp.sum(z3 * z3)), 1.0e-12)
        nt = jnp.maximum(jnp.sqrt(jnp.sum(zt * zt)), 1.0e-12)
        acc_ref[3] += jnp.sum(z3 * zt) / (n3 * nt)

    out_ref[0, 0] = acc_ref[0]
    out_ref[0, 1] = acc_ref[1]
    out_ref[0, 2] = acc_ref[2]
    out_ref[0, 3] = acc_ref[3]


def _tc_part(P, PT, P0, z_3d, z_text):
    B, N, _ = P.shape
    T = N // _R
    P0T = P0.transpose(0, 2, 1)
    scalar_spec = pl.BlockSpec((1, 4), lambda b, t: (0, 0),
                               memory_space=pltpu.SMEM)
    return pl.pallas_call(
        _tc_body,
        grid=(B, T),
        in_specs=[
            pl.BlockSpec((1, _R, 3), lambda b, t: (b, t, 0)),
            pl.BlockSpec((1, 3, N), lambda b, t: (b, 0, 0)),
            pl.BlockSpec((1, 3, N), lambda b, t: (b, 0, 0)),
            pl.BlockSpec((1, 1, z_3d.shape[1]), lambda b, t: (b, 0, 0)),
            pl.BlockSpec((1, 1, z_text.shape[1]), lambda b, t: (b, 0, 0)),
        ],
        out_specs=[scalar_spec],
        out_shape=[jax.ShapeDtypeStruct((1, 4), jnp.float32)],
        scratch_shapes=[pltpu.VMEM((1, N), jnp.float32),
                        pltpu.SMEM((4,), jnp.float32)],
        compiler_params=pltpu.CompilerParams(
            dimension_semantics=("arbitrary", "arbitrary"),
        ),
    )(P, PT, P0T, z_3d[:, None, :], z_text[:, None, :])


def _make_sc_smooth(B, N):
    M = N - _TC_SMOOTH_ROWS
    PPB = _NW // B
    RPW = M // PPB
    CH = N // _L
    mesh = plsc.VectorSubcoreMesh(core_axis_name="c", subcore_axis_name="s")

    @functools.partial(
        pl.kernel,
        out_type=jax.ShapeDtypeStruct((_NW, _L), jnp.float32),
        mesh=mesh,
        scratch_types=[
            pltpu.VMEM((N,), jnp.float32),
            pltpu.VMEM((N,), jnp.float32),
            pltpu.VMEM((N,), jnp.float32),
            pltpu.VMEM((CH, _L), jnp.float32),
            pltpu.VMEM((_L,), jnp.float32),
        ],
        compiler_params=pltpu.CompilerParams(needs_layout_passes=False),
    )
    def sc_smooth(pt_hbm, out_hbm, pxv, pyv, pzv, dbuf, accv):
        wid = lax.axis_index("s") * _NC + lax.axis_index("c")
        b = wid // PPB
        part = wid % PPB
        pltpu.sync_copy(pt_hbm.at[b * 3], pxv)
        pltpu.sync_copy(pt_hbm.at[b * 3 + 1], pyv)
        pltpu.sync_copy(pt_hbm.at[b * 3 + 2], pzv)
        base = _TC_SMOOTH_ROWS + part * RPW

        lane_ids = jnp.arange(_L, dtype=jnp.int32)
        inf = jnp.full((_L,), jnp.inf)

        def row_body(r, acc):
            g0 = base + (r // _L) * _L
            onehot = lane_ids == jnp.full((_L,), r % _L, dtype=jnp.int32)
            gx = pxv[pl.ds(g0, _L)]
            gy = pyv[pl.ds(g0, _L)]
            gz = pzv[pl.ds(g0, _L)]
            qx = jnp.full((_L,), jnp.sum(jnp.where(onehot, gx, 0.0)))
            qy = jnp.full((_L,), jnp.sum(jnp.where(onehot, gy, 0.0)))
            qz = jnp.full((_L,), jnp.sum(jnp.where(onehot, gz, 0.0)))

            pxv[pl.ds(g0, _L)] = jnp.where(onehot, inf, gx)

            def ch_body(c, ts):
                kx = pxv[pl.ds(c * _L, _L)]
                ky = pyv[pl.ds(c * _L, _L)]
                kz = pzv[pl.ds(c * _L, _L)]
                dx = qx - kx
                dy = qy - ky
                dz = qz - kz
                d = dx * dx + dy * dy + dz * dz
                dbuf[c] = d
                x = d
                out = []
                for t in ts:
                    lo = jnp.minimum(t, x)
                    x = jnp.maximum(t, x)
                    out.append(lo)
                return tuple(out)

            ts = lax.fori_loop(0, CH, ch_body, (inf,) * _K)

            work = list(ts)
            th = None
            for i in range(_K):
                m = work[0]
                for w in work[1:]:
                    m = jnp.minimum(m, w)
                s = jnp.min(m)
                if i < _K - 1:
                    sv = jnp.full((_L,), s)
                    work = [jnp.where(w <= sv, jnp.inf, w) for w in work]
                else:
                    th = s
            thv = jnp.full((_L,), th)

            def p2_body(c, carry):
                sx, sy, sz, cnt = carry
                d = dbuf[c]
                kx = pxv[pl.ds(c * _L, _L)]
                ky = pyv[pl.ds(c * _L, _L)]
                kz = pzv[pl.ds(c * _L, _L)]
                msk = d <= thv
                sx = sx + jnp.where(msk, kx, 0.0)
                sy = sy + jnp.where(msk, ky, 0.0)
                sz = sz + jnp.where(msk, kz, 0.0)
                cnt = cnt + jnp.where(msk, 1.0, 0.0)
                return (sx, sy, sz, cnt)

            z = jnp.zeros((_L,), jnp.float32)
            sx, sy, sz, cnt = lax.fori_loop(0, CH, p2_body, (z, z, z, z))
            pxv[pl.ds(g0, _L)] = gx

            csv = jnp.full((_L,), jnp.sum(cnt))
            ex = qx - jnp.full((_L,), jnp.sum(sx)) / csv
            ey = qy - jnp.full((_L,), jnp.sum(sy)) / csv
            ez = qz - jnp.full((_L,), jnp.sum(sz)) / csv
            return acc + ex * ex + ey * ey + ez * ez

        acc = lax.fori_loop(0, RPW, row_body, jnp.zeros((_L,), jnp.float32))
        accv[...] = acc
        pltpu.sync_copy(accv, out_hbm.at[wid])

    return sc_smooth


@jax.jit
def kernel(P, P0, z_3d, z_text):
    B, N, _ = P.shape
    PT = P.transpose(0, 2, 1)
    (tc_out,) = _tc_part(P, PT, P0, z_3d, z_text)
    sc_out = _make_sc_smooth(B, N)(PT.reshape(B * 3, N))

    inv = 1.0 / (B * N)
    L_stick = (tc_out[0, 0] + tc_out[0, 1]) * inv
    L_smooth = (tc_out[0, 2] + jnp.sum(sc_out[:, 0])) * (inv / 3.0)
    L_text = -tc_out[0, 3] / B
    L_total = (_LAMBDA_TEXT * L_text + _LAMBDA_STICK * L_stick
               + _LAMBDA_SMOOTH * L_smooth)
    return jnp.stack([L_total, L_text, L_stick, L_smooth])


# speedup vs baseline: 1.4254x; 1.0525x over previous
import functools
import jax
import jax.numpy as jnp
from jax import lax
from jax.experimental import pallas as pl
from jax.experimental.pallas import tpu as pltpu, tpu_sc as plsc

_K = 8
_LAMBDA_TEXT = 0.5
_LAMBDA_STICK = 2.0
_LAMBDA_SMOOTH = 0.1

_NC, _NS, _L = 2, 16, 16
_NW = _NC * _NS

_TC_SMOOTH_ROWS = 1536
_R = 256


def _tc_body(p_rows_ref, pt_ref, p0t_ref, z3_ref, zt_ref,
             out_ref, colmin_ref, acc_ref, facc_ref, sacc_ref):
    b = pl.program_id(0)
    t = pl.program_id(1)
    T = pl.num_programs(1)
    B = pl.num_programs(0)
    p_tile = p_rows_ref[0]
    pt = pt_ref[0]
    p0t = p0t_ref[0]
    R = p_tile.shape[0]
    N = pt.shape[1]

    px = p_tile[:, 0:1]
    py = p_tile[:, 1:2]
    pz = p_tile[:, 2:3]
    dx = px - p0t[0:1, :]
    dy = py - p0t[1:2, :]
    dz = pz - p0t[2:3, :]
    d0 = dx * dx + dy * dy + dz * dz

    @pl.when((b == 0) & (t == 0))
    def _():
        acc_ref[1] = 0.0
        acc_ref[3] = 0.0
        facc_ref[...] = jnp.zeros_like(facc_ref)
        sacc_ref[...] = jnp.zeros_like(sacc_ref)

    facc_ref[...] += jnp.min(d0, axis=1, keepdims=True)
    cm = jnp.min(d0, axis=0, keepdims=True)

    @pl.when(t == 0)
    def _():
        colmin_ref[...] = cm

    @pl.when(t > 0)
    def _():
        colmin_ref[...] = jnp.minimum(colmin_ref[...], cm)

    @pl.when(t == T - 1)
    def _():
        acc_ref[1] += jnp.sum(colmin_ref[...])

    @pl.when(t < _TC_SMOOTH_ROWS // R)
    def _():
        sx = px - pt[0:1, :]
        sy = py - pt[1:2, :]
        sz = pz - pt[2:3, :]
        ds = sx * sx + sy * sy + sz * sz
        row_ids = t * R + lax.broadcasted_iota(jnp.int32, (R, N), 0)
        col_ids = lax.broadcasted_iota(jnp.int32, (R, N), 1)
        ds = jnp.where(row_ids == col_ids, ds + 1.0e6, ds)

        work = ds
        for _ in range(_K - 1):
            m = jnp.min(work, axis=1, keepdims=True)
            work = jnp.where(work <= m, jnp.inf, work)
        t8 = jnp.min(work, axis=1, keepdims=True)

        maskf = (ds <= t8).astype(jnp.float32)
        count = jnp.sum(maskf, axis=1, keepdims=True)
        nx = jnp.sum(maskf * pt[0:1, :], axis=1, keepdims=True)
        ny = jnp.sum(maskf * pt[1:2, :], axis=1, keepdims=True)
        nz = jnp.sum(maskf * pt[2:3, :], axis=1, keepdims=True)
        ex = px - nx / count
        ey = py - ny / count
        ez = pz - nz / count
        sacc_ref[...] += ex * ex + ey * ey + ez * ez

    @pl.when(t == 0)
    def _():
        z3 = z3_ref[...]
        zt = zt_ref[...]
        n3 = jnp.maximum(jnp.sqrt(jnp.sum(z3 * z3)), 1.0e-12)
        nt = jnp.maximum(jnp.sqrt(jnp.sum(zt * zt)), 1.0e-12)
        acc_ref[3] += jnp.sum(z3 * zt) / (n3 * nt)

    @pl.when((b == B - 1) & (t == T - 1))
    def _():
        out_ref[0, 0] = jnp.sum(facc_ref[...])
        out_ref[0, 1] = acc_ref[1]
        out_ref[0, 2] = jnp.sum(sacc_ref[...])
        out_ref[0, 3] = acc_ref[3]


def _tc_part(P, PT, P0, z_3d, z_text):
    B, N, _ = P.shape
    T = N // _R
    P0T = P0.transpose(0, 2, 1)
    scalar_spec = pl.BlockSpec((1, 4), lambda b, t: (0, 0),
                               memory_space=pltpu.SMEM)
    return pl.pallas_call(
        _tc_body,
        grid=(B, T),
        in_specs=[
            pl.BlockSpec((1, _R, 3), lambda b, t: (b, t, 0)),
            pl.BlockSpec((1, 3, N), lambda b, t: (b, 0, 0)),
            pl.BlockSpec((1, 3, N), lambda b, t: (b, 0, 0)),
            pl.BlockSpec((1, 1, z_3d.shape[1]), lambda b, t: (b, 0, 0)),
            pl.BlockSpec((1, 1, z_text.shape[1]), lambda b, t: (b, 0, 0)),
        ],
        out_specs=[scalar_spec],
        out_shape=[jax.ShapeDtypeStruct((1, 4), jnp.float32)],
        scratch_shapes=[pltpu.VMEM((1, N), jnp.float32),
                        pltpu.SMEM((4,), jnp.float32),
                        pltpu.VMEM((_R, 1), jnp.float32),
                        pltpu.VMEM((_R, 1), jnp.float32)],
        compiler_params=pltpu.CompilerParams(
            dimension_semantics=("arbitrary", "arbitrary"),
        ),
    )(P, PT, P0T, z_3d[:, None, :], z_text[:, None, :])


def _make_sc_smooth(B, N):
    M = N - _TC_SMOOTH_ROWS
    PPB = _NW // B
    RPW = M // PPB
    CH = N // _L
    mesh = plsc.VectorSubcoreMesh(core_axis_name="c", subcore_axis_name="s")

    @functools.partial(
        pl.kernel,
        out_type=jax.ShapeDtypeStruct((_NW, _L), jnp.float32),
        mesh=mesh,
        scratch_types=[
            pltpu.VMEM((N,), jnp.float32),
            pltpu.VMEM((N,), jnp.float32),
            pltpu.VMEM((N,), jnp.float32),
            pltpu.VMEM((CH, _L), jnp.float32),
            pltpu.VMEM((_L,), jnp.float32),
        ],
        compiler_params=pltpu.CompilerParams(needs_layout_passes=False),
    )
    def sc_smooth(pt_hbm, out_hbm, pxv, pyv, pzv, dbuf, accv):
        wid = lax.axis_index("s") * _NC + lax.axis_index("c")
        b = wid // PPB
        part = wid % PPB
        pltpu.sync_copy(pt_hbm.at[b * 3], pxv)
        pltpu.sync_copy(pt_hbm.at[b * 3 + 1], pyv)
        pltpu.sync_copy(pt_hbm.at[b * 3 + 2], pzv)
        base = _TC_SMOOTH_ROWS + part * RPW

        lane_ids = jnp.arange(_L, dtype=jnp.int32)
        inf = jnp.full((_L,), jnp.inf)

        def row_body(r, acc):
            g0 = base + (r // _L) * _L
            onehot = lane_ids == jnp.full((_L,), r % _L, dtype=jnp.int32)
            gx = pxv[pl.ds(g0, _L)]
            gy = pyv[pl.ds(g0, _L)]
            gz = pzv[pl.ds(g0, _L)]
            qx = jnp.full((_L,), jnp.sum(jnp.where(onehot, gx, 0.0)))
            qy = jnp.full((_L,), jnp.sum(jnp.where(onehot, gy, 0.0)))
            qz = jnp.full((_L,), jnp.sum(jnp.where(onehot, gz, 0.0)))

            pxv[pl.ds(g0, _L)] = jnp.where(onehot, inf, gx)

            def ch_body(c, ts):
                kx = pxv[pl.ds(c * _L, _L)]
                ky = pyv[pl.ds(c * _L, _L)]
                kz = pzv[pl.ds(c * _L, _L)]
                dx = qx - kx
                dy = qy - ky
                dz = qz - kz
                d = dx * dx + dy * dy + dz * dz
                dbuf[c] = d
                x = d
                out = []
                for t in ts:
                    lo = jnp.minimum(t, x)
                    x = jnp.maximum(t, x)
                    out.append(lo)
                return tuple(out)

            ts = lax.fori_loop(0, CH, ch_body, (inf,) * _K)

            work = list(ts)
            th = None
            for i in range(_K):
                m = work[0]
                for w in work[1:]:
                    m = jnp.minimum(m, w)
                s = jnp.min(m)
                if i < _K - 1:
                    sv = jnp.full((_L,), s)
                    work = [jnp.where(w <= sv, jnp.inf, w) for w in work]
                else:
                    th = s
            thv = jnp.full((_L,), th)

            def p2_body(c, carry):
                sx, sy, sz, cnt = carry
                d = dbuf[c]
                kx = pxv[pl.ds(c * _L, _L)]
                ky = pyv[pl.ds(c * _L, _L)]
                kz = pzv[pl.ds(c * _L, _L)]
                msk = d <= thv
                sx = sx + jnp.where(msk, kx, 0.0)
                sy = sy + jnp.where(msk, ky, 0.0)
                sz = sz + jnp.where(msk, kz, 0.0)
                cnt = cnt + jnp.where(msk, 1.0, 0.0)
                return (sx, sy, sz, cnt)

            z = jnp.zeros((_L,), jnp.float32)
            sx, sy, sz, cnt = lax.fori_loop(0, CH, p2_body, (z, z, z, z))
            pxv[pl.ds(g0, _L)] = gx

            csv = jnp.full((_L,), jnp.sum(cnt))
            ex = qx - jnp.full((_L,), jnp.sum(sx)) / csv
            ey = qy - jnp.full((_L,), jnp.sum(sy)) / csv
            ez = qz - jnp.full((_L,), jnp.sum(sz)) / csv
            return acc + ex * ex + ey * ey + ez * ez

        acc = lax.fori_loop(0, RPW, row_body, jnp.zeros((_L,), jnp.float32))
        accv[...] = acc
        pltpu.sync_copy(accv, out_hbm.at[wid])

    return sc_smooth


@jax.jit
def kernel(P, P0, z_3d, z_text):
    B, N, _ = P.shape
    PT = P.transpose(0, 2, 1)
    (tc_out,) = _tc_part(P, PT, P0, z_3d, z_text)
    sc_out = _make_sc_smooth(B, N)(PT.reshape(B * 3, N))

    inv = 1.0 / (B * N)
    L_stick = (tc_out[0, 0] + tc_out[0, 1]) * inv
    L_smooth = (tc_out[0, 2] + jnp.sum(sc_out[:, 0])) * (inv / 3.0)
    L_text = -tc_out[0, 3] / B
    L_total = (_LAMBDA_TEXT * L_text + _LAMBDA_STICK * L_stick
               + _LAMBDA_SMOOTH * L_smooth)
    return jnp.stack([L_total, L_text, L_stick, L_smooth])
